# Initial kernel scaffold; baseline (speedup 1.0000x reference)
import jax
import jax.numpy as jnp
from jax.experimental import pallas as pl


def kernel(e1, e2, soil_table, cover_table):
    raise NotImplementedError("write your pallas kernel here")


# speedup vs baseline: 4.3859x; 4.3859x over previous
import functools

import jax
import jax.numpy as jnp
from jax import lax
from jax.experimental import pallas as pl
from jax.experimental.pallas import tpu as pltpu
from jax.experimental.pallas import tpu_sc as plsc

B, L, D = 16384, 20, 128
N = B * L
NW = 32
PER_W = N // NW
CHUNK = 128
NCHUNK = PER_W // CHUNK
LANES = 16

_mesh = plsc.VectorSubcoreMesh(core_axis_name="c", subcore_axis_name="s")


@functools.partial(
    pl.kernel,
    out_type=jax.ShapeDtypeStruct((N, D), jnp.float32),
    mesh=_mesh,
    scratch_types=[
        pltpu.VMEM((CHUNK,), jnp.int32),
        pltpu.VMEM((CHUNK,), jnp.int32),
        pltpu.VMEM((CHUNK,), jnp.int32),
        pltpu.VMEM((CHUNK, D), jnp.float32),
        pltpu.SemaphoreType.DMA,
    ],
)
def _emb_kernel(e1_hbm, e2_hbm, comb_hbm, out_hbm, e1_v, e2_v, idx_v, rows_v, sem):
    wid = lax.axis_index("s") * 2 + lax.axis_index("c")
    w_base = wid * PER_W

    def chunk_body(c, carry):
        base = w_base + c * CHUNK
        pltpu.sync_copy(e1_hbm.at[pl.ds(base, CHUNK)], e1_v)
        pltpu.sync_copy(e2_hbm.at[pl.ds(base, CHUNK)], e2_v)
        for k in range(CHUNK // LANES):
            s = pl.ds(k * LANES, LANES)
            idx_v[s] = e1_v[s] * 16 + e2_v[s]
        pltpu.async_copy(comb_hbm.at[idx_v], rows_v, sem).wait()
        pltpu.sync_copy(rows_v, out_hbm.at[pl.ds(base, CHUNK)])
        return carry

    lax.fori_loop(0, NCHUNK, chunk_body, 0)


def kernel(e1, e2, soil_table, cover_table):
    e1f = e1.reshape(-1).astype(jnp.int32)
    e2f = e2.reshape(-1).astype(jnp.int32)
    comb = jnp.concatenate(
        [jnp.repeat(soil_table, 16, axis=0), jnp.tile(cover_table, (12, 1))],
        axis=1,
    )
    out = _emb_kernel(e1f, e2f, comb)
    return out.reshape(B, L, D)


# speedup vs baseline: 4.5042x; 1.0270x over previous
import functools

import jax
import jax.numpy as jnp
from jax import lax
from jax.experimental import pallas as pl
from jax.experimental.pallas import tpu as pltpu
from jax.experimental.pallas import tpu_sc as plsc

B, L, D = 16384, 20, 128
N = B * L
NW = 32
PER_W = N // NW
CHUNK = 256
NBLK = CHUNK // 128
NCHUNK = PER_W // CHUNK
LANES = 16

_mesh = plsc.VectorSubcoreMesh(core_axis_name="c", subcore_axis_name="s")


@functools.partial(
    pl.kernel,
    out_type=jax.ShapeDtypeStruct((N, D), jnp.float32),
    mesh=_mesh,
    scratch_types=[
        pltpu.VMEM((2, CHUNK), jnp.int32),
        pltpu.VMEM((2, CHUNK), jnp.int32),
        pltpu.VMEM((2, NBLK, 128), jnp.int32),
        pltpu.VMEM((2, CHUNK, D), jnp.float32),
        [pltpu.SemaphoreType.DMA] * 2,
        [pltpu.SemaphoreType.DMA] * 2,
        [pltpu.SemaphoreType.DMA] * 2,
    ],
)
def _emb_kernel(e1_hbm, e2_hbm, comb_hbm, out_hbm,
                e1_v, e2_v, idx_v, rows_v, sem_l, sem_g, sem_w):
    wid = lax.axis_index("s") * 2 + lax.axis_index("c")
    w_base = wid * PER_W

    def start_load(c, b):
        base = w_base + c * CHUNK
        pltpu.async_copy(e1_hbm.at[pl.ds(base, CHUNK)], e1_v.at[b], sem_l[b])
        pltpu.async_copy(e2_hbm.at[pl.ds(base, CHUNK)], e2_v.at[b], sem_l[b])

    def wait_load(b):
        pltpu.make_async_copy(e1_hbm.at[pl.ds(0, CHUNK)], e1_v.at[b], sem_l[b]).wait()
        pltpu.make_async_copy(e2_hbm.at[pl.ds(0, CHUNK)], e2_v.at[b], sem_l[b]).wait()

    def compute_idx(b):
        e1_b, e2_b = e1_v.at[b], e2_v.at[b]
        for blk in range(NBLK):
            idx_b = idx_v.at[b].at[blk]
            for k in range(128 // LANES):
                s = pl.ds(k * LANES, LANES)
                sf = pl.ds(blk * 128 + k * LANES, LANES)
                idx_b[s] = e1_b[sf] * 16 + e2_b[sf]

    def start_gather(b):
        for blk in range(NBLK):
            pltpu.async_copy(comb_hbm.at[idx_v.at[b].at[blk]],
                             rows_v.at[b].at[pl.ds(blk * 128, 128)], sem_g[b])

    def wait_gather(b):
        for blk in range(NBLK):
            pltpu.make_async_copy(comb_hbm.at[idx_v.at[b].at[blk]],
                                  rows_v.at[b].at[pl.ds(blk * 128, 128)],
                                  sem_g[b]).wait()

    def start_write(c, b):
        base = w_base + c * CHUNK
        pltpu.async_copy(rows_v.at[b], out_hbm.at[pl.ds(base, CHUNK)], sem_w[b])

    def wait_write(b):
        pltpu.make_async_copy(rows_v.at[b], out_hbm.at[pl.ds(0, CHUNK)],
                              sem_w[b]).wait()

    start_load(0, 0)
    for b in range(2):
        wait_load(b)
        compute_idx(b)
        start_gather(b)
        start_load(b + 1, b ^ 1)
        wait_gather(b)
        start_write(b, b)

    def pair_body(p, carry):
        for b in range(2):
            c = 2 * p + b
            wait_load(b)
            compute_idx(b)
            wait_write(b)
            start_gather(b)

            @pl.when(c + 1 < NCHUNK)
            def _():
                start_load(c + 1, b ^ 1)

            wait_gather(b)
            start_write(c, b)
        return carry

    lax.fori_loop(1, NCHUNK // 2, pair_body, 0)
    for b in range(2):
        wait_write(b)


def kernel(e1, e2, soil_table, cover_table):
    e1f = e1.reshape(-1).astype(jnp.int32)
    e2f = e2.reshape(-1).astype(jnp.int32)
    comb = jnp.concatenate(
        [jnp.repeat(soil_table, 16, axis=0), jnp.tile(cover_table, (12, 1))],
        axis=1,
    )
    out = _emb_kernel(e1f, e2f, comb)
    return out.reshape(B, L, D)


# speedup vs baseline: 4.6916x; 1.0416x over previous
import functools

import jax
import jax.numpy as jnp
from jax import lax
from jax.experimental import pallas as pl
from jax.experimental.pallas import tpu as pltpu
from jax.experimental.pallas import tpu_sc as plsc

B, L, D = 16384, 20, 128
N = B * L
NW = 32
ROWS_W = B // NW
PER_W = N // NW
CHUNK = 128
NCHUNK = PER_W // CHUNK
LANES = 16
GRP = CHUNK // LANES

_mesh = plsc.VectorSubcoreMesh(core_axis_name="c", subcore_axis_name="s")


@functools.partial(
    pl.kernel,
    out_type=jax.ShapeDtypeStruct((N, D), jnp.float32),
    mesh=_mesh,
    scratch_types=[
        pltpu.VMEM((PER_W,), jnp.int32),
        pltpu.VMEM((2, 1, CHUNK), jnp.int32),
        pltpu.VMEM((2, CHUNK, D), jnp.float32),
        pltpu.SemaphoreType.DMA,
        [pltpu.SemaphoreType.DMA] * 2,
        [pltpu.SemaphoreType.DMA] * 2,
    ],
)
def _emb_kernel(idx_hbm, comb_hbm, out_hbm,
                idxblk_v, idx_v, rows_v, sem_l, sem_g, sem_w):
    wid = lax.axis_index("s") * 2 + lax.axis_index("c")
    w_base = wid * PER_W

    pltpu.sync_copy(idx_hbm.at[pl.ds(w_base, PER_W)], idxblk_v)

    def compute_idx(c, b):
        idx_b = idx_v.at[b].at[0]
        for k in range(GRP):
            s = pl.ds(k * LANES, LANES)
            sf = pl.ds(c * CHUNK + k * LANES, LANES)
            idx_b[s] = idxblk_v[sf]

    def start_gather(b):
        pltpu.async_copy(comb_hbm.at[idx_v.at[b].at[0]], rows_v.at[b], sem_g[b])

    def wait_gather(b):
        pltpu.make_async_copy(comb_hbm.at[idx_v.at[b].at[0]], rows_v.at[b],
                              sem_g[b]).wait()

    def start_write(c, b):
        base = w_base + c * CHUNK
        pltpu.async_copy(rows_v.at[b], out_hbm.at[pl.ds(base, CHUNK)], sem_w[b])

    def wait_write(b):
        pltpu.make_async_copy(rows_v.at[b], out_hbm.at[pl.ds(0, CHUNK)],
                              sem_w[b]).wait()

    for b in range(2):
        compute_idx(b, b)
        start_gather(b)
        wait_gather(b)
        start_write(b, b)

    def pair_body(p, carry):
        for b in range(2):
            c = 2 * p + b
            compute_idx(c, b)
            wait_write(b)
            start_gather(b)
            wait_gather(b)
            start_write(c, b)
        return carry

    lax.fori_loop(1, NCHUNK // 2, pair_body, 0)
    for b in range(2):
        wait_write(b)


def kernel(e1, e2, soil_table, cover_table):
    comb = jnp.concatenate(
        [jnp.repeat(soil_table, 16, axis=0), jnp.tile(cover_table, (12, 1))],
        axis=1,
    )
    idxf = (e1.astype(jnp.int32) * 16 + e2.astype(jnp.int32)).reshape(-1)
    out = _emb_kernel(idxf, comb)
    return out.reshape(B, L, D)


# speedup vs baseline: 7.6981x; 1.6408x over previous
import functools

import jax
import jax.numpy as jnp
from jax import lax
from jax.experimental import pallas as pl
from jax.experimental.pallas import tpu as pltpu
from jax.experimental.pallas import tpu_sc as plsc

B, L, D = 16384, 20, 128
N = B * L
NW = 32
ROWS_W = B // NW
PER_W = N // NW
CHUNK = 128
NCHUNK = PER_W // CHUNK
LANES = 16
GRP = CHUNK // LANES

_mesh = plsc.VectorSubcoreMesh(core_axis_name="c", subcore_axis_name="s")


@functools.partial(
    pl.kernel,
    out_type=jax.ShapeDtypeStruct((N, D), jnp.float32),
    mesh=_mesh,
    scratch_types=[
        pltpu.VMEM((PER_W,), jnp.int32),
        pltpu.VMEM((2, 1, CHUNK), jnp.int32),
        pltpu.VMEM((2, CHUNK, D), jnp.float32),
        pltpu.VMEM_SHARED((192, D), jnp.float32),
        pltpu.SemaphoreType.DMA,
        [pltpu.SemaphoreType.DMA] * 2,
        [pltpu.SemaphoreType.DMA] * 2,
    ],
)
def _emb_kernel(idx_hbm, comb_hbm, out_hbm,
                idxblk_v, idx_v, rows_v, comb_sh, sem_l, sem_g, sem_w):
    wid = lax.axis_index("s") * 2 + lax.axis_index("c")
    w_base = wid * PER_W

    @pl.when(lax.axis_index("s") == 0)
    def _():
        pltpu.sync_copy(comb_hbm, comb_sh)

    pltpu.sync_copy(idx_hbm.at[pl.ds(w_base, PER_W)], idxblk_v)
    plsc.subcore_barrier()

    def compute_idx(c, b):
        idx_b = idx_v.at[b].at[0]
        for k in range(GRP):
            s = pl.ds(k * LANES, LANES)
            sf = pl.ds(c * CHUNK + k * LANES, LANES)
            idx_b[s] = idxblk_v[sf]

    def start_gather(b):
        pltpu.async_copy(comb_sh.at[idx_v.at[b].at[0]], rows_v.at[b], sem_g[b])

    def wait_gather(b):
        pltpu.make_async_copy(comb_sh.at[idx_v.at[b].at[0]], rows_v.at[b],
                              sem_g[b]).wait()

    def start_write(c, b):
        base = w_base + c * CHUNK
        pltpu.async_copy(rows_v.at[b], out_hbm.at[pl.ds(base, CHUNK)], sem_w[b])

    def wait_write(b):
        pltpu.make_async_copy(rows_v.at[b], out_hbm.at[pl.ds(0, CHUNK)],
                              sem_w[b]).wait()

    for b in range(2):
        compute_idx(b, b)
        start_gather(b)
        wait_gather(b)
        start_write(b, b)

    def pair_body(p, carry):
        for b in range(2):
            c = 2 * p + b
            compute_idx(c, b)
            wait_write(b)
            start_gather(b)
            wait_gather(b)
            start_write(c, b)
        return carry

    lax.fori_loop(1, NCHUNK // 2, pair_body, 0)
    for b in range(2):
        wait_write(b)


def kernel(e1, e2, soil_table, cover_table):
    comb = jnp.concatenate(
        [jnp.repeat(soil_table, 16, axis=0), jnp.tile(cover_table, (12, 1))],
        axis=1,
    )
    idxf = (e1.astype(jnp.int32) * 16 + e2.astype(jnp.int32)).reshape(-1)
    out = _emb_kernel(idxf, comb)
    return out.reshape(B, L, D)


# speedup vs baseline: 13.9670x; 1.8143x over previous
import functools

import jax
import jax.numpy as jnp
from jax import lax
from jax.experimental import pallas as pl
from jax.experimental.pallas import tpu as pltpu
from jax.experimental.pallas import tpu_sc as plsc

B, L, D = 16384, 20, 128
N = B * L
NW = 32
ROWS_W = B // NW
PER_W = N // NW
RB = 4
CHUNK = RB * L
NCHUNK = ROWS_W // RB
LANES = 16
GRP = CHUNK // LANES

_mesh = plsc.VectorSubcoreMesh(core_axis_name="c", subcore_axis_name="s")


@functools.partial(
    pl.kernel,
    out_type=jax.ShapeDtypeStruct((B, L, D), jnp.float32),
    mesh=_mesh,
    compiler_params=pltpu.CompilerParams(use_tc_tiling_on_sc=True),
    scratch_types=[
        pltpu.VMEM((PER_W,), jnp.int32),
        pltpu.VMEM((2, 1, CHUNK), jnp.int32),
        pltpu.VMEM((2, CHUNK, D), jnp.float32),
        pltpu.VMEM_SHARED((192, D), jnp.float32),
        [pltpu.SemaphoreType.DMA] * 2,
        [pltpu.SemaphoreType.DMA] * 2,
    ],
)
def _emb_kernel(idx_hbm, comb_hbm, out_hbm,
                idxblk_v, idx_v, rows_v, comb_sh, sem_g, sem_w):
    wid = lax.axis_index("s") * 2 + lax.axis_index("c")
    w_base = wid * PER_W
    row0 = wid * ROWS_W

    @pl.when(lax.axis_index("s") == 0)
    def _():
        pltpu.sync_copy(comb_hbm, comb_sh)

    pltpu.sync_copy(idx_hbm.at[pl.ds(w_base, PER_W)], idxblk_v)
    plsc.subcore_barrier()

    def compute_idx(c, b):
        idx_b = idx_v.at[b].at[0]
        for k in range(GRP):
            s = pl.ds(k * LANES, LANES)
            sf = pl.ds(c * CHUNK + k * LANES, LANES)
            idx_b[s] = idxblk_v[sf]

    def start_gather(b):
        pltpu.async_copy(comb_sh.at[idx_v.at[b].at[0]], rows_v.at[b], sem_g[b])

    def wait_gather(b):
        pltpu.make_async_copy(comb_sh.at[idx_v.at[b].at[0]], rows_v.at[b],
                              sem_g[b]).wait()

    def start_write(c, b):
        for j in range(RB):
            pltpu.async_copy(rows_v.at[b].at[pl.ds(j * L, L)],
                             out_hbm.at[row0 + c * RB + j], sem_w[b])

    def wait_write(b):
        for j in range(RB):
            pltpu.make_async_copy(rows_v.at[b].at[pl.ds(j * L, L)],
                                  out_hbm.at[row0], sem_w[b]).wait()

    for b in range(2):
        compute_idx(b, b)
        start_gather(b)
        wait_gather(b)
        start_write(b, b)

    def pair_body(p, carry):
        for b in range(2):
            c = 2 * p + b
            compute_idx(c, b)
            wait_write(b)
            start_gather(b)
            wait_gather(b)
            start_write(c, b)
        return carry

    lax.fori_loop(1, NCHUNK // 2, pair_body, 0)
    for b in range(2):
        wait_write(b)


def kernel(e1, e2, soil_table, cover_table):
    comb = jnp.concatenate(
        [jnp.repeat(soil_table, 16, axis=0), jnp.tile(cover_table, (12, 1))],
        axis=1,
    )
    idxf = (e1.astype(jnp.int32) * 16 + e2.astype(jnp.int32)).reshape(-1)
    return _emb_kernel(idxf, comb)
